# speedup vs baseline: 4.4447x; 4.4447x over previous
import jax
import jax.numpy as jnp
from jax.experimental import pallas as pl

_M = 8192
_N = 16384
_C = 128
_NQ = 128

_VS = (0.05, 0.05, 0.1)


def _nn_interp_kernel(q_ref, xiT_ref, feat_ref, out_ref):
    xiT = xiT_ref[...].astype(jnp.float32)
    kb = xiT[0:1, :]
    kx = xiT[3:4, :] * _VS[0] + (1.5 * _VS[0])
    ky = xiT[2:3, :] * _VS[1] + (1.5 * _VS[1])
    kz = xiT[1:2, :] * _VS[2] + (1.5 * _VS[2])
    kk = kb * kb + kx * kx + ky * ky + kz * kz

    q = q_ref[...]
    qq = jnp.sum(q * q, axis=1, keepdims=True)
    cross = (q[:, 0:1] * kb + q[:, 1:2] * kx
             + q[:, 2:3] * ky + q[:, 3:4] * kz)
    d2 = jnp.maximum(qq + kk - 2.0 * cross, 0.0)

    lane = jax.lax.broadcasted_iota(jnp.int32, (_NQ, _M), 1)
    big = jnp.float32(1e30)

    def min_argmin(d):
        m = jnp.min(d, axis=1, keepdims=True)
        i = jnp.min(jnp.where(d == m, lane, _M), axis=1, keepdims=True)
        return m, i

    m0, i0 = min_argmin(d2)
    d2 = jnp.where(lane == i0, big, d2)
    m1, i1 = min_argmin(d2)
    d2 = jnp.where(lane == i1, big, d2)
    m2, i2 = min_argmin(d2)

    r0 = 1.0 / (m0 + 1e-8)
    r1 = 1.0 / (m1 + 1e-8)
    r2 = 1.0 / (m2 + 1e-8)
    norm = r0 + r1 + r2
    w0 = r0 / norm
    w1 = r1 / norm
    w2 = r2 / norm

    w = (jnp.where(lane == i0, w0, 0.0)
         + jnp.where(lane == i1, w1, 0.0)
         + jnp.where(lane == i2, w2, 0.0))

    out_ref[...] = jnp.dot(w, feat_ref[...],
                           preferred_element_type=jnp.float32,
                           precision=jax.lax.Precision.HIGHEST)


def kernel(x_features, x_indices, points_mean):
    xiT = x_indices.astype(jnp.int32).T

    grid = (_N // _NQ,)
    out = pl.pallas_call(
        _nn_interp_kernel,
        grid=grid,
        in_specs=[
            pl.BlockSpec((_NQ, 4), lambda i: (i, 0)),
            pl.BlockSpec((4, _M), lambda i: (0, 0)),
            pl.BlockSpec((_M, _C), lambda i: (0, 0)),
        ],
        out_specs=pl.BlockSpec((_NQ, _C), lambda i: (i, 0)),
        out_shape=jax.ShapeDtypeStruct((_N, _C), jnp.float32),
    )(points_mean, xiT, x_features)
    return out


# speedup vs baseline: 4.7779x; 1.0750x over previous
import jax
import jax.numpy as jnp
from jax.experimental import pallas as pl

_M = 8192
_N = 16384
_C = 128
_NQ = 128

_VS = (0.05, 0.05, 0.1)


def _nn_interp_kernel(q_ref, xiT_ref, feat_ref, out_ref):
    xiT = xiT_ref[...].astype(jnp.float32)
    kb = xiT[0:1, :]
    kx = xiT[3:4, :] * _VS[0] + (1.5 * _VS[0])
    ky = xiT[2:3, :] * _VS[1] + (1.5 * _VS[1])
    kz = xiT[1:2, :] * _VS[2] + (1.5 * _VS[2])
    kk = kb * kb + kx * kx + ky * ky + kz * kz

    q = q_ref[...]
    qq = jnp.sum(q * q, axis=1, keepdims=True)
    cross = (q[:, 0:1] * kb + q[:, 1:2] * kx
             + q[:, 2:3] * ky + q[:, 3:4] * kz)
    d2 = jnp.maximum((qq + kk) - 2.0 * cross, 0.0)

    lane = jax.lax.broadcasted_iota(jnp.int32, (_NQ, _M), 1).astype(jnp.float32)
    big = jnp.float32(1e30)
    bigl = jnp.float32(_M)

    def pass_j(d, mask_after):
        m = jnp.min(d, axis=1, keepdims=True)
        i = jnp.min(jnp.where(d == m, lane, bigl), axis=1, keepdims=True)
        eqm = lane == i
        r = 1.0 / (m + 1e-8)
        acc_j = jnp.where(eqm, r, 0.0)
        if mask_after:
            d = jnp.where(eqm, big, d)
        return d, r, acc_j

    d2, r0, a0 = pass_j(d2, True)
    d2, r1, a1 = pass_j(d2, True)
    _, r2, a2 = pass_j(d2, False)

    inv_norm = 1.0 / (r0 + r1 + r2)
    w = (a0 + a1 + a2) * inv_norm

    out_ref[...] = jnp.dot(w, feat_ref[...],
                           preferred_element_type=jnp.float32,
                           precision=jax.lax.Precision.HIGHEST)


def kernel(x_features, x_indices, points_mean):
    xiT = x_indices.astype(jnp.int32).T

    grid = (_N // _NQ,)
    out = pl.pallas_call(
        _nn_interp_kernel,
        grid=grid,
        in_specs=[
            pl.BlockSpec((_NQ, 4), lambda i: (i, 0)),
            pl.BlockSpec((4, _M), lambda i: (0, 0)),
            pl.BlockSpec((_M, _C), lambda i: (0, 0)),
        ],
        out_specs=pl.BlockSpec((_NQ, _C), lambda i: (i, 0)),
        out_shape=jax.ShapeDtypeStruct((_N, _C), jnp.float32),
    )(points_mean, xiT, x_features)
    return out
